# baseline (speedup 1.0000x reference)
import jax
import jax.numpy as jnp
from jax.experimental import pallas as pl

N = 10000
K = 32
C = 16
NB = 25
BN = N // NB


def _norm(x, axis=-1, eps=1e-12):
    n = jnp.sqrt(jnp.sum(x * x, axis=axis, keepdims=True) + 1e-24)
    return x / jnp.maximum(n, eps)


def _sumsq_body(x_ref, o_ref):
    @pl.when(pl.program_id(0) == 0)
    def _():
        o_ref[...] = jnp.zeros_like(o_ref)

    x = x_ref[...]
    o_ref[...] += jnp.sum(x * x, axis=0, keepdims=True)


def _scale_body(x_ref, ss_ref, o_ref):
    nrm = jnp.sqrt(ss_ref[...] + 1e-24)
    o_ref[...] = x_ref[...] / jnp.maximum(nrm, 1e-12)


def _pallas_norm_axis0(h2):
    ss = pl.pallas_call(
        _sumsq_body,
        grid=(NB,),
        in_specs=[pl.BlockSpec((BN, K * C), lambda i: (i, 0))],
        out_specs=pl.BlockSpec((1, K * C), lambda i: (0, 0)),
        out_shape=jax.ShapeDtypeStruct((1, K * C), jnp.float32),
    )(h2)
    out = pl.pallas_call(
        _scale_body,
        grid=(NB,),
        in_specs=[
            pl.BlockSpec((BN, K * C), lambda i: (i, 0)),
            pl.BlockSpec((1, K * C), lambda i: (0, 0)),
        ],
        out_specs=pl.BlockSpec((BN, K * C), lambda i: (i, 0)),
        out_shape=jax.ShapeDtypeStruct((N, K * C), jnp.float32),
    )(h2, ss)
    return out


def _gather_nodes(nodes, idx):
    Bn, Nn, Kn = idx.shape
    flat = idx.reshape(Bn, Nn * Kn)
    g = jnp.take_along_axis(nodes, flat[:, :, None], axis=1)
    return g.reshape(Bn, Nn, Kn, nodes.shape[-1])


def _quaternions(R):
    diag = jnp.diagonal(R, axis1=-2, axis2=-1)
    Rxx, Ryy, Rzz = diag[..., 0], diag[..., 1], diag[..., 2]
    magnitudes = 0.5 * jnp.sqrt(jnp.abs(1.0 + jnp.stack(
        [Rxx - Ryy - Rzz, -Rxx + Ryy - Rzz, -Rxx - Ryy + Rzz], -1)))
    signs = jnp.sign(jnp.stack(
        [R[..., 2, 1] - R[..., 1, 2], R[..., 0, 2] - R[..., 2, 0],
         R[..., 1, 0] - R[..., 0, 1]], -1))
    xyz = signs * magnitudes
    w = jnp.sqrt(jax.nn.relu(1.0 + jnp.sum(diag, -1, keepdims=True))) / 2.0
    Q = jnp.concatenate([xyz, w], -1)
    return _norm(Q, -1)


def kernel(p, e_idx, mask, t_frq, t_s, W, b):
    X = p
    dX = X[:, 1:, :] - X[:, :-1, :]
    U = _norm(dX, -1)
    u_2 = U[:, :-2, :]
    u_1 = U[:, 1:-1, :]
    n_2 = _norm(jnp.cross(u_2, u_1), -1)
    o_1 = _norm(u_2 - u_1, -1)
    O = jnp.stack([o_1, n_2, jnp.cross(o_1, n_2)], axis=2)
    O = O.reshape(O.shape[0], O.shape[1], 9)
    O = jnp.pad(O, ((0, 0), (1, 2), (0, 0)))
    O_neighbours = _gather_nodes(O, e_idx)
    X_neighbours = _gather_nodes(X, e_idx)
    O = O.reshape(O.shape[0], O.shape[1], 3, 3)
    O_neighbours = O_neighbours.reshape(
        O_neighbours.shape[0], O_neighbours.shape[1], O_neighbours.shape[2], 3, 3)
    dXn = X_neighbours - X[:, :, None, :]
    dU = jnp.matmul(O[:, :, None], dXn[..., None])[..., 0]
    dU = _norm(dU, -1)
    R = jnp.matmul(jnp.swapaxes(O[:, :, None], -1, -2), O_neighbours)
    Q = _quaternions(R)
    q_feats = jnp.concatenate([dU, Q], -1)

    ii = jnp.arange(N, dtype=jnp.float32).reshape(1, -1, 1)
    d = (e_idx.astype(jnp.float32) - ii)[..., None]
    FR = t_frq * t_s
    angles = d * FR.reshape(1, 1, 1, -1)
    e_pe = jnp.concatenate([jnp.sin(angles), jnp.cos(angles)], -1)
    e_cat = jnp.concatenate([e_pe, q_feats], -1)
    h_e = jnp.matmul(e_cat, W.T) + b

    h2 = h_e.reshape(N, K * C)
    out = _pallas_norm_axis0(h2)
    return out.reshape(1, N, K, C)


# speedup vs baseline: 2.2063x; 2.2063x over previous
import functools
import jax
import jax.numpy as jnp
from jax import lax
from jax.experimental import pallas as pl
from jax.experimental.pallas import tpu as pltpu, tpu_sc as plsc

N = 10000
K = 32
C = 16
NB = 25
BN = N // NB

E = N * K
D = 16
NW = 32
B_PER_W = E // NW
CH = 2000


def _sc_gather(table, idx_flat):
    mesh = plsc.VectorSubcoreMesh(core_axis_name="c", subcore_axis_name="s")

    @functools.partial(
        pl.kernel,
        mesh=mesh,
        out_type=jax.ShapeDtypeStruct((E, D), jnp.float32),
        scratch_types=[
            pltpu.VMEM((CH,), jnp.int32),
            pltpu.VMEM((CH, D), jnp.float32),
            pltpu.SemaphoreType.DMA,
        ],
        compiler_params=pltpu.CompilerParams(use_tc_tiling_on_sc=False),
    )
    def gather_k(table_hbm, idx_hbm, out_hbm, idx_v, rows_v, sem):
        wid = lax.axis_index("s") * 2 + lax.axis_index("c")
        base = wid * B_PER_W
        for c in range(B_PER_W // CH):
            off = base + c * CH
            pltpu.sync_copy(idx_hbm.at[pl.ds(off, CH)], idx_v)
            pltpu.async_copy(table_hbm.at[idx_v], rows_v, sem).wait()
            pltpu.sync_copy(rows_v, out_hbm.at[pl.ds(off, CH)])

    return gather_k(table, idx_flat)


def _norm(x, axis=-1, eps=1e-12):
    n = jnp.sqrt(jnp.sum(x * x, axis=axis, keepdims=True) + 1e-24)
    return x / jnp.maximum(n, eps)


def _sumsq_body(x_ref, o_ref):
    @pl.when(pl.program_id(0) == 0)
    def _():
        o_ref[...] = jnp.zeros_like(o_ref)

    x = x_ref[...]
    o_ref[...] += jnp.sum(x * x, axis=0, keepdims=True)


def _scale_body(x_ref, ss_ref, o_ref):
    nrm = jnp.sqrt(ss_ref[...] + 1e-24)
    o_ref[...] = x_ref[...] / jnp.maximum(nrm, 1e-12)


def _pallas_norm_axis0(h2):
    ss = pl.pallas_call(
        _sumsq_body,
        grid=(NB,),
        in_specs=[pl.BlockSpec((BN, K * C), lambda i: (i, 0))],
        out_specs=pl.BlockSpec((1, K * C), lambda i: (0, 0)),
        out_shape=jax.ShapeDtypeStruct((1, K * C), jnp.float32),
    )(h2)
    out = pl.pallas_call(
        _scale_body,
        grid=(NB,),
        in_specs=[
            pl.BlockSpec((BN, K * C), lambda i: (i, 0)),
            pl.BlockSpec((1, K * C), lambda i: (0, 0)),
        ],
        out_specs=pl.BlockSpec((BN, K * C), lambda i: (i, 0)),
        out_shape=jax.ShapeDtypeStruct((N, K * C), jnp.float32),
    )(h2, ss)
    return out


def _gather_nodes(nodes, idx):
    Bn, Nn, Kn = idx.shape
    flat = idx.reshape(Bn, Nn * Kn)
    g = jnp.take_along_axis(nodes, flat[:, :, None], axis=1)
    return g.reshape(Bn, Nn, Kn, nodes.shape[-1])


def _quaternions(R):
    diag = jnp.diagonal(R, axis1=-2, axis2=-1)
    Rxx, Ryy, Rzz = diag[..., 0], diag[..., 1], diag[..., 2]
    magnitudes = 0.5 * jnp.sqrt(jnp.abs(1.0 + jnp.stack(
        [Rxx - Ryy - Rzz, -Rxx + Ryy - Rzz, -Rxx - Ryy + Rzz], -1)))
    signs = jnp.sign(jnp.stack(
        [R[..., 2, 1] - R[..., 1, 2], R[..., 0, 2] - R[..., 2, 0],
         R[..., 1, 0] - R[..., 0, 1]], -1))
    xyz = signs * magnitudes
    w = jnp.sqrt(jax.nn.relu(1.0 + jnp.sum(diag, -1, keepdims=True))) / 2.0
    Q = jnp.concatenate([xyz, w], -1)
    return _norm(Q, -1)


def kernel(p, e_idx, mask, t_frq, t_s, W, b):
    X = p
    dX = X[:, 1:, :] - X[:, :-1, :]
    U = _norm(dX, -1)
    u_2 = U[:, :-2, :]
    u_1 = U[:, 1:-1, :]
    n_2 = _norm(jnp.cross(u_2, u_1), -1)
    o_1 = _norm(u_2 - u_1, -1)
    O = jnp.stack([o_1, n_2, jnp.cross(o_1, n_2)], axis=2)
    O = O.reshape(O.shape[0], O.shape[1], 9)
    O = jnp.pad(O, ((0, 0), (1, 2), (0, 0)))
    table = jnp.concatenate(
        [O[0], X[0], jnp.zeros((N, D - 12), jnp.float32)], axis=-1)
    g = _sc_gather(table, e_idx.reshape(E))
    O_neighbours = g[:, :9].reshape(1, N, K, 9)
    X_neighbours = g[:, 9:12].reshape(1, N, K, 3)
    O = O.reshape(O.shape[0], O.shape[1], 3, 3)
    O_neighbours = O_neighbours.reshape(
        O_neighbours.shape[0], O_neighbours.shape[1], O_neighbours.shape[2], 3, 3)
    dXn = X_neighbours - X[:, :, None, :]
    dU = jnp.matmul(O[:, :, None], dXn[..., None])[..., 0]
    dU = _norm(dU, -1)
    R = jnp.matmul(jnp.swapaxes(O[:, :, None], -1, -2), O_neighbours)
    Q = _quaternions(R)
    q_feats = jnp.concatenate([dU, Q], -1)

    ii = jnp.arange(N, dtype=jnp.float32).reshape(1, -1, 1)
    d = (e_idx.astype(jnp.float32) - ii)[..., None]
    FR = t_frq * t_s
    angles = d * FR.reshape(1, 1, 1, -1)
    e_pe = jnp.concatenate([jnp.sin(angles), jnp.cos(angles)], -1)
    e_cat = jnp.concatenate([e_pe, q_feats], -1)
    h_e = jnp.matmul(e_cat, W.T) + b

    h2 = h_e.reshape(N, K * C)
    out = _pallas_norm_axis0(h2)
    return out.reshape(1, N, K, C)


# speedup vs baseline: 3.1493x; 1.4274x over previous
import functools
import jax
import jax.numpy as jnp
from jax import lax
from jax.experimental import pallas as pl
from jax.experimental.pallas import tpu as pltpu, tpu_sc as plsc

N = 10000
K = 32
C = 16
E = N * K
D = 16
NW = 32
B_PER_W = E // NW
CH = 2000

NBLK = 25
EB = E // NBLK
SB = EB // 128


def _k1_body(pT_ref, t_ref):
    p0 = pT_ref[0:1, :]
    p1 = pT_ref[1:2, :]
    p2 = pT_ref[2:3, :]

    def norm3(a, b, c):
        inv = 1.0 / jnp.sqrt(a * a + b * b + c * c + 1e-24)
        return a * inv, b * inv, c * inv

    dx = [p0[:, 1:] - p0[:, :-1],
          p1[:, 1:] - p1[:, :-1],
          p2[:, 1:] - p2[:, :-1]]
    U = norm3(*dx)
    u2 = [U[a][:, :-2] for a in range(3)]
    u1 = [U[a][:, 1:-1] for a in range(3)]
    cr = [u2[1] * u1[2] - u2[2] * u1[1],
          u2[2] * u1[0] - u2[0] * u1[2],
          u2[0] * u1[1] - u2[1] * u1[0]]
    n2 = norm3(*cr)
    o1 = norm3(u2[0] - u1[0], u2[1] - u1[1], u2[2] - u1[2])
    o3 = [o1[1] * n2[2] - o1[2] * n2[1],
          o1[2] * n2[0] - o1[0] * n2[2],
          o1[0] * n2[1] - o1[1] * n2[0]]
    rows = list(o1) + list(n2) + o3
    for c in range(9):
        t_ref[c:c + 1, :] = jnp.pad(rows[c], ((0, 0), (1, 2)))
    t_ref[9:10, :] = p0
    t_ref[10:11, :] = p1
    t_ref[11:12, :] = p2
    t_ref[12:16, :] = jnp.zeros((4, N), jnp.float32)


def _build_table(pT):
    return pl.pallas_call(
        _k1_body,
        out_shape=jax.ShapeDtypeStruct((D, N), jnp.float32),
    )(pT)


def _sc_gather(table, idx_nbr, idx_self):
    mesh = plsc.VectorSubcoreMesh(core_axis_name="c", subcore_axis_name="s")

    @functools.partial(
        pl.kernel,
        mesh=mesh,
        out_type=(
            jax.ShapeDtypeStruct((E, D), jnp.float32),
            jax.ShapeDtypeStruct((E, D), jnp.float32),
        ),
        scratch_types=[
            pltpu.VMEM((CH,), jnp.int32),
            pltpu.VMEM((CH,), jnp.int32),
            pltpu.VMEM((CH, D), jnp.float32),
            pltpu.VMEM((CH, D), jnp.float32),
            pltpu.SemaphoreType.DMA,
            pltpu.SemaphoreType.DMA,
        ],
        compiler_params=pltpu.CompilerParams(use_tc_tiling_on_sc=False),
    )
    def gather_k(table_hbm, idxn_hbm, idxs_hbm, outn_hbm, outs_hbm,
                 idxn_v, idxs_v, rown_v, rows_v, semn, sems):
        wid = lax.axis_index("s") * 2 + lax.axis_index("c")
        base = wid * B_PER_W
        for c in range(B_PER_W // CH):
            off = base + c * CH
            pltpu.sync_copy(idxn_hbm.at[pl.ds(off, CH)], idxn_v)
            pltpu.sync_copy(idxs_hbm.at[pl.ds(off, CH)], idxs_v)
            cpn = pltpu.async_copy(table_hbm.at[idxn_v], rown_v, semn)
            cps = pltpu.async_copy(table_hbm.at[idxs_v], rows_v, sems)
            cpn.wait()
            cps.wait()
            pltpu.sync_copy(rown_v, outn_hbm.at[pl.ds(off, CH)])
            pltpu.sync_copy(rows_v, outs_hbm.at[pl.ds(off, CH)])

    return gather_k(table, idx_nbr, idx_self)


def _k3_body(g_ref, s_ref, ei_ref, tfrq_ref, ts_ref, W_ref, b_ref,
             h_ref, ss_ref):
    pid = pl.program_id(0)

    @pl.when(pid == 0)
    def _():
        ss_ref[...] = jnp.zeros_like(ss_ref)

    G = g_ref[0]
    S = s_ref[0]
    oj = [G[i] for i in range(9)]
    pj = [G[9 + a] for a in range(3)]
    on = [S[i] for i in range(9)]
    pn = [S[9 + a] for a in range(3)]

    def b16(x):
        return x.astype(jnp.bfloat16).astype(jnp.float32)

    onb = [b16(v) for v in on]
    ojb = [b16(v) for v in oj]

    dx = [b16(pj[a] - pn[a]) for a in range(3)]
    t = [onb[3 * i] * dx[0] + onb[3 * i + 1] * dx[1] + onb[3 * i + 2] * dx[2]
         for i in range(3)]
    inv = 1.0 / jnp.sqrt(t[0] * t[0] + t[1] * t[1] + t[2] * t[2] + 1e-24)
    dU = [t[i] * inv for i in range(3)]

    def RR(a, b):
        return (onb[a] * ojb[b] + onb[3 + a] * ojb[3 + b]
                + onb[6 + a] * ojb[6 + b])

    Rxx, Ryy, Rzz = RR(0, 0), RR(1, 1), RR(2, 2)
    d21 = RR(2, 1) - RR(1, 2)
    d02 = RR(0, 2) - RR(2, 0)
    d10 = RR(1, 0) - RR(0, 1)
    mx = 0.5 * jnp.sqrt(jnp.abs(1.0 + Rxx - Ryy - Rzz))
    my = 0.5 * jnp.sqrt(jnp.abs(1.0 - Rxx + Ryy - Rzz))
    mz = 0.5 * jnp.sqrt(jnp.abs(1.0 - Rxx - Ryy + Rzz))
    qx = jnp.sign(d21) * mx
    qy = jnp.sign(d02) * my
    qz = jnp.sign(d10) * mz
    qw = 0.5 * jnp.sqrt(jax.nn.relu(1.0 + Rxx + Ryy + Rzz))
    qinv = 1.0 / jnp.sqrt(qx * qx + qy * qy + qz * qz + qw * qw + 1e-24)
    Q = [qx * qinv, qy * qinv, qz * qinv, qw * qinv]

    j = ei_ref[0].astype(jnp.float32)
    si = lax.broadcasted_iota(jnp.int32, (SB, 128), 0)
    li = lax.broadcasted_iota(jnp.int32, (SB, 128), 1)
    n = (pid * (EB // K) + si * (128 // K) + li // K).astype(jnp.float32)
    d = j - n
    ts = ts_ref[0]
    feats = []
    coss = []
    for f in range(8):
        ang = d * (tfrq_ref[f] * ts)
        feats.append(jnp.sin(ang))
        coss.append(jnp.cos(ang))
    feats = feats + coss + dU + Q

    for c in range(C):
        acc = jnp.full((SB, 128), b_ref[c], jnp.float32)
        for f in range(23):
            acc = acc + W_ref[c, f] * feats[f]
        h_ref[0, c] = acc
        ss_ref[c:c + 1, :] += jnp.sum(acc * acc, axis=0, keepdims=True)


def _k3_call(gt4, st4, ei3, t_frq, t_s, W, b):
    return pl.pallas_call(
        _k3_body,
        grid=(NBLK,),
        in_specs=[
            pl.BlockSpec((1, D, SB, 128), lambda i: (i, 0, 0, 0)),
            pl.BlockSpec((1, D, SB, 128), lambda i: (i, 0, 0, 0)),
            pl.BlockSpec((1, SB, 128), lambda i: (i, 0, 0)),
            pl.BlockSpec(memory_space=pltpu.SMEM),
            pl.BlockSpec(memory_space=pltpu.SMEM),
            pl.BlockSpec(memory_space=pltpu.SMEM),
            pl.BlockSpec(memory_space=pltpu.SMEM),
        ],
        out_specs=[
            pl.BlockSpec((1, C, SB, 128), lambda i: (i, 0, 0, 0)),
            pl.BlockSpec((C, 128), lambda i: (0, 0)),
        ],
        out_shape=[
            jax.ShapeDtypeStruct((NBLK, C, SB, 128), jnp.float32),
            jax.ShapeDtypeStruct((C, 128), jnp.float32),
        ],
    )(gt4, st4, ei3, t_frq, t_s, W, b)


def _k4_body(h_ref, ss_ref, o_ref):
    ss = ss_ref[...]
    ssf = ss[:, 0:32] + ss[:, 32:64] + ss[:, 64:96] + ss[:, 96:128]
    nrm = jnp.sqrt(ssf + 1e-24)
    sc = 1.0 / jnp.maximum(nrm, 1e-12)
    scb = jnp.concatenate([sc, sc, sc, sc], axis=1)
    for c in range(C):
        o_ref[0, c] = h_ref[0, c] * jnp.broadcast_to(scb[c:c + 1, :], (SB, 128))


def _k4_call(ht4, ss):
    return pl.pallas_call(
        _k4_body,
        grid=(NBLK,),
        in_specs=[
            pl.BlockSpec((1, C, SB, 128), lambda i: (i, 0, 0, 0)),
            pl.BlockSpec((C, 128), lambda i: (0, 0)),
        ],
        out_specs=pl.BlockSpec((1, C, SB, 128), lambda i: (i, 0, 0, 0)),
        out_shape=jax.ShapeDtypeStruct((NBLK, C, SB, 128), jnp.float32),
    )(ht4, ss)


def kernel(p, e_idx, mask, t_frq, t_s, W, b):
    pT = jnp.transpose(p[0])
    table = jnp.transpose(_build_table(pT))

    ei = e_idx.reshape(E).astype(jnp.int32)
    ids = (lax.iota(jnp.int32, E) // K)
    g, s = _sc_gather(table, ei, ids)

    def to4(x):
        return jnp.transpose(x).reshape(D, NBLK, SB, 128).transpose(1, 0, 2, 3)

    gt4 = to4(g)
    st4 = to4(s)
    ei3 = ei.reshape(NBLK, SB, 128)

    ht4, ss = _k3_call(gt4, st4, ei3, t_frq.astype(jnp.float32),
                       t_s.reshape(1), W, b)
    out4 = _k4_call(ht4, ss)

    out = out4.transpose(1, 0, 2, 3).reshape(C, E)
    return jnp.transpose(out).reshape(1, N, K, C)
